# baseline (speedup 1.0000x reference)
import functools

import jax
import jax.numpy as jnp
from jax import lax
from jax.experimental import pallas as pl
from jax.experimental.pallas import tpu as pltpu
from jax.experimental.pallas import tpu_sc as plsc

_D = 64
_SCALE = 8.0
_NC, _NS = 2, 16
_NW = _NC * _NS
_CHUNK = 512
_NBUF = 2


@functools.partial(jax.jit, static_argnums=(2,))
def _lookup(idx_flat, table, B):
    b_per_w = B // _NW
    nchunks = b_per_w // _CHUNK
    assert nchunks % 2 == 0 and nchunks >= 4

    mesh = plsc.VectorSubcoreMesh(core_axis_name="c", subcore_axis_name="s")

    @functools.partial(
        pl.kernel,
        out_type=jax.ShapeDtypeStruct((B, _D), jnp.float32),
        mesh=mesh,
        scratch_types=[
            pltpu.VMEM((b_per_w,), jnp.int32),
            pltpu.VMEM((_NBUF, _CHUNK, _D), jnp.float32),
            pltpu.SemaphoreType.DMA,
            pltpu.SemaphoreType.DMA,
            pltpu.SemaphoreType.DMA,
            pltpu.SemaphoreType.DMA,
        ],
        compiler_params=pltpu.CompilerParams(use_tc_tiling_on_sc=False),
    )
    def emb(idx_hbm, table_hbm, out_hbm, idx_v, rows_v, g0, g1, s0, s1):
        wid = lax.axis_index("s") * _NC + lax.axis_index("c")
        base = wid * b_per_w
        pltpu.sync_copy(idx_hbm.at[pl.ds(base, b_per_w)], idx_v)

        gsem = (g0, g1)
        ssem = (s0, s1)

        def start_gather(slot, c):
            pltpu.make_async_copy(
                table_hbm.at[idx_v.at[pl.ds(c * _CHUNK, _CHUNK)]],
                rows_v.at[slot],
                gsem[slot],
            ).start()

        def wait_gather(slot):
            pltpu.make_async_copy(
                table_hbm.at[idx_v.at[pl.ds(0, _CHUNK)]],
                rows_v.at[slot],
                gsem[slot],
            ).wait()

        def start_scatter(slot, c):
            pltpu.make_async_copy(
                rows_v.at[slot],
                out_hbm.at[pl.ds(base + c * _CHUNK, _CHUNK)],
                ssem[slot],
            ).start()

        def wait_scatter(slot):
            pltpu.make_async_copy(
                rows_v.at[slot],
                out_hbm.at[pl.ds(base, _CHUNK)],
                ssem[slot],
            ).wait()

        def scale(slot):
            @pl.loop(0, _CHUNK, unroll=4)
            def _(r):
                for j in range(_D // 16):
                    sl = (slot, r, pl.ds(j * 16, 16))
                    rows_v[sl] = rows_v[sl] * _SCALE

        start_gather(0, 0)
        start_gather(1, 1)

        @pl.loop(0, (nchunks - 2) // 2)
        def _(i):
            c0 = i * 2
            for b in range(_NBUF):
                wait_gather(b)
                scale(b)
                start_scatter(b, c0 + b)
            for b in range(_NBUF):
                wait_scatter(b)
                start_gather(b, c0 + 2 + b)

        for b in range(_NBUF):
            wait_gather(b)
            scale(b)
            start_scatter(b, nchunks - 2 + b)
        for b in range(_NBUF):
            wait_scatter(b)

    return emb(idx_flat, table)


def kernel(x, table):
    batch, seq = x.shape
    out = _lookup(x.reshape(-1).astype(jnp.int32), table, batch * seq)
    return out.reshape(batch, seq, _D)
